# Initial kernel scaffold; baseline (speedup 1.0000x reference)
import jax
import jax.numpy as jnp
from jax import lax
from jax.experimental import pallas as pl
from jax.experimental.pallas import tpu as pltpu
from jax.experimental.pallas import tpu_sc as plsc

F_DIM = 8192
SEQ_LEN = 20
BATCH = 1024

_NC = 2
_NS = 16
_NW = _NC * _NS
_COLS = BATCH // _NW
_L = 16


def _sc_body(x_hbm, w_hbm, b_hbm, neg_hbm, pos_hbm, w_v, x_v, b_v, neg_v,
             pos_v):
    wid = lax.axis_index("s") * _NC + lax.axis_index("c")
    base = wid * _COLS

    pltpu.sync_copy(w_hbm, w_v)
    pltpu.sync_copy(b_hbm, b_v)
    pltpu.sync_copy(x_hbm.at[:, pl.ds(base, _COLS)], x_v)

    bias = b_v[...]
    for g in range(_COLS // _L):
        xs = [x_v[l, pl.ds(g * _L, _L)] for l in range(SEQ_LEN)]
        acc = bias
        for l in range(SEQ_LEN):
            skip = xs[l] == 0
            for j in range(l):
                skip = skip | (xs[l] == xs[j])
            w_val = plsc.load_gather(w_v, [xs[l]])
            acc = acc + jnp.where(skip, jnp.zeros((_L,), jnp.float32), w_val)
        pos = jnp.sign(acc)
        pos_v[pl.ds(g * _L, _L)] = pos
        neg_v[pl.ds(g * _L, _L)] = -pos

    pltpu.sync_copy(neg_v, neg_hbm.at[pl.ds(base, _COLS)])
    pltpu.sync_copy(pos_v, pos_hbm.at[pl.ds(base, _COLS)])


@jax.jit
def _nb_scores(x, w_flat, b16):
    run = pl.kernel(
        _sc_body,
        out_type=(
            jax.ShapeDtypeStruct((BATCH,), jnp.float32),
            jax.ShapeDtypeStruct((BATCH,), jnp.float32),
        ),
        scratch_types=[
            pltpu.VMEM((F_DIM,), jnp.float32),
            pltpu.VMEM((SEQ_LEN, _COLS), jnp.int32),
            pltpu.VMEM((_L,), jnp.float32),
            pltpu.VMEM((_COLS,), jnp.float32),
            pltpu.VMEM((_COLS,), jnp.float32),
        ],
        mesh=plsc.VectorSubcoreMesh(core_axis_name="c", subcore_axis_name="s"),
    )
    return run(x, w_flat, b16)


def kernel(x, E, W, b):
    del E
    w_flat = W.reshape(F_DIM)
    b16 = jnp.broadcast_to(b.astype(jnp.float32), (_L,))
    neg, pos = _nb_scores(x, w_flat, b16)
    return jnp.stack([neg, pos], axis=1)


# speedup vs baseline: 29.7933x; 29.7933x over previous
import jax
import jax.numpy as jnp
from jax import lax
from jax.experimental import pallas as pl
from jax.experimental.pallas import tpu as pltpu
from jax.experimental.pallas import tpu_sc as plsc

F_DIM = 8192
SEQ_LEN = 20
BATCH = 1024

_NC = 2
_NS = 16
_NW = _NC * _NS
_COLS = BATCH // _NW
_L = 16


def _sc_body(x_hbm, w_hbm, b_hbm, neg_hbm, pos_hbm, w_v, x_v, b_v, neg_v,
             pos_v):
    wid = lax.axis_index("s") * _NC + lax.axis_index("c")
    base = wid * _COLS

    pltpu.sync_copy(w_hbm, w_v)
    pltpu.sync_copy(b_hbm, b_v)
    pltpu.sync_copy(x_hbm.at[pl.ds(wid * SEQ_LEN * _COLS, SEQ_LEN * _COLS)],
                    x_v)

    bias = b_v[...]
    for g in range(_COLS // _L):
        xs = [x_v[pl.ds(l * _COLS + g * _L, _L)] for l in range(SEQ_LEN)]
        acc = bias
        for l in range(SEQ_LEN):
            skip = xs[l] == 0
            for j in range(l):
                skip = skip | (xs[l] == xs[j])
            w_val = plsc.load_gather(w_v, [xs[l]])
            acc = acc + jnp.where(skip, jnp.zeros((_L,), jnp.float32), w_val)
        pos = jnp.sign(acc)
        pos_v[pl.ds(g * _L, _L)] = pos
        neg_v[pl.ds(g * _L, _L)] = -pos

    pltpu.sync_copy(neg_v, neg_hbm.at[pl.ds(base, _COLS)])
    pltpu.sync_copy(pos_v, pos_hbm.at[pl.ds(base, _COLS)])


@jax.jit
def _nb_scores(x, w_flat, b16):
    run = pl.kernel(
        _sc_body,
        out_type=(
            jax.ShapeDtypeStruct((BATCH,), jnp.float32),
            jax.ShapeDtypeStruct((BATCH,), jnp.float32),
        ),
        scratch_types=[
            pltpu.VMEM((F_DIM,), jnp.float32),
            pltpu.VMEM((SEQ_LEN * _COLS,), jnp.int32),
            pltpu.VMEM((_L,), jnp.float32),
            pltpu.VMEM((_COLS,), jnp.float32),
            pltpu.VMEM((_COLS,), jnp.float32),
        ],
        mesh=plsc.VectorSubcoreMesh(core_axis_name="c", subcore_axis_name="s"),
        compiler_params=pltpu.CompilerParams(needs_layout_passes=False),
    )
    return run(x, w_flat, b16)


def kernel(x, E, W, b):
    del E
    x_r = x.reshape(SEQ_LEN, _NW, _COLS).transpose(1, 0, 2).reshape(-1)
    w_flat = lax.reduce_precision(W.reshape(F_DIM), exponent_bits=8,
                                  mantissa_bits=7)
    b16 = jnp.broadcast_to(b.astype(jnp.float32), (_L,))
    neg, pos = _nb_scores(x_r, w_flat, b16)
    return jnp.stack([neg, pos], axis=1)


# speedup vs baseline: 32.3876x; 1.0871x over previous
import jax
import jax.numpy as jnp
from jax import lax
from jax.experimental import pallas as pl
from jax.experimental.pallas import tpu as pltpu
from jax.experimental.pallas import tpu_sc as plsc

F_DIM = 8192
SEQ_LEN = 20
BATCH = 1024

_NC = 2
_NS = 16
_NW = _NC * _NS
_COLS = BATCH // _NW
_L = 16
_G = _COLS // _L


def _round_bf16(v):
    u = plsc.bitcast(v, jnp.int32)
    r = (u + jnp.int32(32767) + ((u >> 16) & 1)) & jnp.int32(-65536)
    return plsc.bitcast(r, jnp.float32)


def _sc_body(x_hbm, w_hbm, b_hbm, out_hbm, w_v, x_v, b_v, out_v, sem_w,
             sem_x):
    wid = lax.axis_index("s") * _NC + lax.axis_index("c")
    base = wid * _COLS

    cp_w = pltpu.async_copy(w_hbm, w_v, sem_w)
    cp_b = pltpu.async_copy(b_hbm, b_v.at[pl.ds(0, 1)], sem_x)
    cp_x = [
        pltpu.async_copy(x_hbm.at[pl.ds(l * BATCH + base, _COLS)],
                         x_v.at[pl.ds(l * _COLS, _COLS)], sem_x)
        for l in range(SEQ_LEN)
    ]
    for c in cp_x:
        c.wait()
    cp_b.wait()

    xs = [[x_v[pl.ds(l * _COLS + g * _L, _L)] for l in range(SEQ_LEN)]
          for g in range(_G)]
    skips = []
    for g in range(_G):
        sk = []
        for l in range(SEQ_LEN):
            s = xs[g][l] == 0
            for j in range(l):
                s = s | (xs[g][l] == xs[g][j])
            sk.append(s)
        skips.append(sk)

    cp_w.wait()
    zero = jnp.zeros((_L,), jnp.float32)
    bias = jnp.full((_L,), b_v[pl.ds(0, _L)][0], jnp.float32)
    for g in range(_G):
        acc = zero
        for l in range(SEQ_LEN):
            w_val = _round_bf16(plsc.load_gather(w_v, [xs[g][l]]))
            acc = acc + jnp.where(skips[g][l], zero, w_val)
        pos = jnp.sign(acc + bias)
        lr = lax.iota(jnp.int32, _L) + jnp.int32(g * _L)
        plsc.store_scatter(out_v, [2 * lr], -pos)
        plsc.store_scatter(out_v, [2 * lr + 1], pos)

    pltpu.sync_copy(out_v, out_hbm.at[pl.ds(wid * 2 * _COLS, 2 * _COLS)])


@jax.jit
def _nb_scores(x_flat, w_flat, b):
    run = pl.kernel(
        _sc_body,
        out_type=jax.ShapeDtypeStruct((2 * BATCH,), jnp.float32),
        scratch_types=[
            pltpu.VMEM((F_DIM,), jnp.float32),
            pltpu.VMEM((SEQ_LEN * _COLS,), jnp.int32),
            pltpu.VMEM((_L,), jnp.float32),
            pltpu.VMEM((2 * _COLS,), jnp.float32),
            pltpu.SemaphoreType.DMA,
            pltpu.SemaphoreType.DMA,
        ],
        mesh=plsc.VectorSubcoreMesh(core_axis_name="c", subcore_axis_name="s"),
        compiler_params=pltpu.CompilerParams(needs_layout_passes=False),
    )
    return run(x_flat, w_flat, b)


def kernel(x, E, W, b):
    del E
    out = _nb_scores(x.reshape(-1), W.reshape(F_DIM), b.astype(jnp.float32))
    return out.reshape(BATCH, 2)


# speedup vs baseline: 32.4996x; 1.0035x over previous
import jax
import jax.numpy as jnp
from jax import lax
from jax.experimental import pallas as pl
from jax.experimental.pallas import tpu as pltpu
from jax.experimental.pallas import tpu_sc as plsc

F_DIM = 8192
SEQ_LEN = 20
BATCH = 1024

_NC = 2
_NS = 16
_NW = _NC * _NS
_COLS = BATCH // _NW
_L = 16
_G = _COLS // _L


def _round_bf16(v):
    u = plsc.bitcast(v, jnp.int32)
    r = (u + jnp.int32(32767) + ((u >> 16) & 1)) & jnp.int32(-65536)
    return plsc.bitcast(r, jnp.float32)


def _sc_body(x_hbm, w_hbm, b_hbm, out_hbm, w_v, x_v, b_v, out_v, sem_w,
             sem_x):
    wid = lax.axis_index("s") * _NC + lax.axis_index("c")
    base = wid * _COLS

    cp_w = pltpu.async_copy(w_hbm, w_v, sem_w)
    cp_b = pltpu.async_copy(b_hbm, b_v.at[pl.ds(0, 1)], sem_x)
    cp_x = [
        pltpu.async_copy(x_hbm.at[pl.ds(l * BATCH + base, _COLS)],
                         x_v.at[pl.ds(l * _COLS, _COLS)], sem_x)
        for l in range(SEQ_LEN)
    ]
    for c in cp_x:
        c.wait()
    cp_b.wait()

    xs = [[x_v[pl.ds(l * _COLS + g * _L, _L)] for l in range(SEQ_LEN)]
          for g in range(_G)]
    skips = []
    for g in range(_G):
        sk = []
        for l in range(SEQ_LEN):
            s = xs[g][l] == 0
            for j in range(l):
                s = s | (xs[g][l] == xs[g][j])
            sk.append(s)
        skips.append(sk)

    cp_w.wait()
    zero = jnp.zeros((_L,), jnp.float32)
    bias = jnp.full((_L,), b_v[pl.ds(0, _L)][0], jnp.float32)
    for g in range(_G):
        acc = zero
        for l in range(SEQ_LEN):
            w_val = _round_bf16(plsc.load_gather(w_v, [xs[g][l]]))
            acc = acc + jnp.where(skips[g][l], zero, w_val)
        pos = jnp.sign(acc + bias)
        lr = lax.iota(jnp.int32, _L) + jnp.int32(g * _L)
        plsc.store_scatter(out_v, [2 * lr], -pos)
        plsc.store_scatter(out_v, [2 * lr + 1], pos)

    pltpu.sync_copy(out_v, out_hbm.at[pl.ds(wid * 2 * _COLS, 2 * _COLS)])


@jax.jit
def _nb_scores(x_flat, w_flat, b):
    run = pl.kernel(
        _sc_body,
        out_type=jax.ShapeDtypeStruct((2 * BATCH,), jnp.float32),
        scratch_types=[
            pltpu.VMEM((F_DIM,), jnp.float32),
            pltpu.VMEM((SEQ_LEN * _COLS,), jnp.int32),
            pltpu.VMEM((_L,), jnp.float32),
            pltpu.VMEM((2 * _COLS,), jnp.float32),
            pltpu.SemaphoreType.DMA,
            pltpu.SemaphoreType.DMA,
        ],
        mesh=plsc.VectorSubcoreMesh(core_axis_name="c", subcore_axis_name="s"),
        compiler_params=pltpu.CompilerParams(needs_layout_passes=False,
                                             skip_device_barrier=True),
    )
    return run(x_flat, w_flat, b)


def kernel(x, E, W, b):
    del E
    out = _nb_scores(x.reshape(-1), W.reshape(F_DIM), b.astype(jnp.float32))
    return out.reshape(BATCH, 2)
